# baseline (speedup 1.0000x reference)
import functools

import jax
import jax.numpy as jnp
from jax import lax
from jax.experimental import pallas as pl
from jax.experimental.pallas import tpu as pltpu
from jax.experimental.pallas import tpu_sc as plsc

NC = 2
NS = 16
NW = NC * NS
LANES = 16
WIN = 128

_f32 = jnp.float32


def _mesh():
  return plsc.VectorSubcoreMesh(core_axis_name="c", subcore_axis_name="s")


def _make_deg(NPAD, NWIN):
  RPT = NPAD // NS

  @functools.partial(
      pl.kernel,
      out_type=jax.ShapeDtypeStruct((NC * NPAD,), _f32),
      mesh=_mesh(),
      scratch_types=[
          pltpu.VMEM((NWIN, WIN), jnp.int32),
          pltpu.VMEM((WIN,), _f32),
          pltpu.VMEM((RPT,), _f32),
          pltpu.VMEM_SHARED((NPAD,), _f32),
      ],
  )
  def deg(dst_hbm, out_hbm, idst, ones_v, zv, dacc):
    c = lax.axis_index("c")
    s = lax.axis_index("s")
    wid = s * NC + c
    for j in range(WIN // LANES):
      ones_v[pl.ds(j * LANES, LANES)] = jnp.ones((LANES,), _f32)
    for j in range(RPT // LANES):
      zv[pl.ds(j * LANES, LANES)] = jnp.zeros((LANES,), _f32)
    pltpu.sync_copy(zv, dacc.at[pl.ds(s * RPT, RPT)])
    pltpu.sync_copy(dst_hbm.at[wid], idst)
    plsc.subcore_barrier()

    def body(w, carry):
      pltpu.sync_copy(ones_v, dacc.at[idst.at[w]], add=True)
      return carry

    lax.fori_loop(0, NWIN, body, 0)
    plsc.subcore_barrier()
    pltpu.sync_copy(dacc.at[pl.ds(s * RPT, RPT)],
                    out_hbm.at[pl.ds(c * NPAD + s * RPT, RPT)])

  return deg


def _make_prop(D, NPAD, NWIN):
  RPT = NPAD // NS
  NCH = 2
  CW = NWIN // NCH

  @functools.partial(
      pl.kernel,
      out_type=jax.ShapeDtypeStruct((NC, NPAD, D), _f32),
      mesh=_mesh(),
      scratch_types=[
          pltpu.VMEM((CW, WIN), jnp.int32),
          pltpu.VMEM((CW, WIN), jnp.int32),
          pltpu.VMEM((WIN, D), _f32),
          pltpu.VMEM((WIN, D), _f32),
          pltpu.VMEM_SHARED((NPAD, D), _f32),
          pltpu.SemaphoreType.DMA,
          pltpu.SemaphoreType.DMA,
          pltpu.SemaphoreType.DMA,
          pltpu.SemaphoreType.DMA,
      ],
  )
  def prop(hs_hbm, zeros_hbm, src_hbm, dst_hbm, out_hbm,
           isrc, idst, rows0, rows1, acc, gsem0, gsem1, ssem0, ssem1):
    c = lax.axis_index("c")
    s = lax.axis_index("s")
    wid = s * NC + c
    r0 = s * RPT
    pltpu.sync_copy(zeros_hbm.at[pl.ds(r0, RPT)], acc.at[pl.ds(r0, RPT)])
    plsc.subcore_barrier()

    def chunk_body(ch, carry):
      pltpu.sync_copy(src_hbm.at[wid, pl.ds(ch * CW, CW)], isrc)
      pltpu.sync_copy(dst_hbm.at[wid, pl.ds(ch * CW, CW)], idst)
      pltpu.async_copy(hs_hbm.at[isrc.at[0]], rows0, gsem0)
      pltpu.async_copy(hs_hbm.at[isrc.at[1]], rows1, gsem1)

      def body(w2, c2):
        w = 2 * w2
        pltpu.make_async_copy(hs_hbm.at[isrc.at[w]], rows0, gsem0).wait()
        pltpu.async_copy(rows0, acc.at[idst.at[w]], ssem0, add=True)
        pltpu.make_async_copy(hs_hbm.at[isrc.at[w + 1]], rows1, gsem1).wait()
        pltpu.async_copy(rows1, acc.at[idst.at[w + 1]], ssem1, add=True)

        pltpu.make_async_copy(rows0, acc.at[idst.at[w]], ssem0).wait()

        @pl.when(w2 + 1 < CW // 2)
        def _():
          pltpu.async_copy(hs_hbm.at[isrc.at[w + 2]], rows0, gsem0)

        pltpu.make_async_copy(rows1, acc.at[idst.at[w + 1]], ssem1).wait()

        @pl.when(w2 + 1 < CW // 2)
        def _():
          pltpu.async_copy(hs_hbm.at[isrc.at[w + 3]], rows1, gsem1)

        return c2

      lax.fori_loop(0, CW // 2, body, 0)
      return carry

    lax.fori_loop(0, NCH, chunk_body, 0)
    plsc.subcore_barrier()
    pltpu.sync_copy(acc.at[pl.ds(r0, RPT)], out_hbm.at[c, pl.ds(r0, RPT)])

  return prop


def _row_block(N):
  for g in (8, 5, 4, 2, 1):
    if N % g == 0 and (N // g) % 8 == 0:
      return N // g
  return N


def _tc1_body(d0, d1, x, w, dinv_o, hs_o):
  dv = lax.rsqrt(d0[...] + d1[...] + 1.0)
  dinv_o[...] = dv
  hs_o[...] = jnp.dot(x[...], w[...], preferred_element_type=_f32) * dv


def _tc2_body(a0, a1, hs, dv, w, hs2_o):
  g = (a0[...] + a1[...] + hs[...]) * dv[...]
  g2 = g * g
  hs2_o[...] = jnp.dot(g2, w[...], preferred_element_type=_f32) * dv[...]


def _tc3_body(a0, a1, hs, dv, out_o):
  out_o[...] = (a0[...] + a1[...] + hs[...]) * dv[...]


def kernel(x, edge_index, W1, W2):
  N, D = x.shape
  E = edge_index.shape[1]

  NWIN = -(-E // (NW * WIN))
  NWIN += NWIN % 2
  EPAD = NW * NWIN * WIN
  RPT = (-(-(N + 1) // NS) + 127) // 128 * 128
  NPAD = NS * RPT
  GR = NPAD - N

  src = edge_index[0].astype(jnp.int32)
  dst = edge_index[1].astype(jnp.int32)
  pad = EPAD - E
  padi = jnp.arange(pad, dtype=jnp.int32)
  src3 = jnp.concatenate([src, padi % N]).reshape(NW, NWIN, WIN)
  dst3 = jnp.concatenate([dst, N + padi % GR]).reshape(NW, NWIN, WIN)
  zeros2 = jnp.zeros((NPAD, D), _f32)

  degflat = _make_deg(NPAD, NWIN)(dst3)
  d0 = degflat[:N, None]
  d1 = degflat[NPAD:NPAD + N, None]

  BLK = _row_block(N)
  G = N // BLK
  colspec = pl.BlockSpec((BLK, 1), lambda i: (i, 0))
  matspec = pl.BlockSpec((BLK, D), lambda i: (i, 0))
  wspec = pl.BlockSpec((D, D), lambda i: (0, 0))

  dinv, hs1 = pl.pallas_call(
      _tc1_body,
      grid=(G,),
      in_specs=[colspec, colspec, matspec, wspec],
      out_specs=[colspec, matspec],
      out_shape=[
          jax.ShapeDtypeStruct((N, 1), _f32),
          jax.ShapeDtypeStruct((N, D), _f32),
      ],
  )(d0, d1, x, W1)

  prop = _make_prop(D, NPAD, NWIN)

  acc1 = prop(hs1, zeros2, src3, dst3)
  hs2 = pl.pallas_call(
      _tc2_body,
      grid=(G,),
      in_specs=[matspec, matspec, matspec, colspec, wspec],
      out_specs=matspec,
      out_shape=jax.ShapeDtypeStruct((N, D), _f32),
  )(acc1[0, :N], acc1[1, :N], hs1, dinv, W2)

  acc2 = prop(hs2, zeros2, src3, dst3)
  out = pl.pallas_call(
      _tc3_body,
      grid=(G,),
      in_specs=[matspec, matspec, matspec, colspec],
      out_specs=matspec,
      out_shape=jax.ShapeDtypeStruct((N, D), _f32),
  )(acc2[0, :N], acc2[1, :N], hs2, dinv)
  return out


# speedup vs baseline: 1.2556x; 1.2556x over previous
import functools

import jax
import jax.numpy as jnp
from jax import lax
from jax.experimental import pallas as pl
from jax.experimental.pallas import tpu as pltpu
from jax.experimental.pallas import tpu_sc as plsc

NC = 2
NS = 16
NW = NC * NS
LANES = 16
WIN = 128

_f32 = jnp.float32


def _mesh():
  return plsc.VectorSubcoreMesh(core_axis_name="c", subcore_axis_name="s")


def _make_deg(NPAD, NWIN):
  RPT = NPAD // NS

  @functools.partial(
      pl.kernel,
      out_type=jax.ShapeDtypeStruct((NC * NPAD,), _f32),
      mesh=_mesh(),
      scratch_types=[
          pltpu.VMEM((NWIN, WIN), jnp.int32),
          pltpu.VMEM((WIN,), _f32),
          pltpu.VMEM((RPT,), _f32),
          pltpu.VMEM_SHARED((NPAD,), _f32),
      ],
  )
  def deg(dst_hbm, out_hbm, idst, ones_v, zv, dacc):
    c = lax.axis_index("c")
    s = lax.axis_index("s")
    wid = s * NC + c
    for j in range(WIN // LANES):
      ones_v[pl.ds(j * LANES, LANES)] = jnp.ones((LANES,), _f32)
    for j in range(RPT // LANES):
      zv[pl.ds(j * LANES, LANES)] = jnp.zeros((LANES,), _f32)
    pltpu.sync_copy(zv, dacc.at[pl.ds(s * RPT, RPT)])
    pltpu.sync_copy(dst_hbm.at[wid], idst)
    plsc.subcore_barrier()

    def body(w, carry):
      pltpu.sync_copy(ones_v, dacc.at[idst.at[w]], add=True)
      return carry

    lax.fori_loop(0, NWIN, body, 0)
    plsc.subcore_barrier()
    pltpu.sync_copy(dacc.at[pl.ds(s * RPT, RPT)],
                    out_hbm.at[pl.ds(c * NPAD + s * RPT, RPT)])

  return deg


def _make_prop(D, NPAD, NWIN):
  RPT = NPAD // NS
  NCH = 2
  CW = NWIN // NCH

  @functools.partial(
      pl.kernel,
      out_type=jax.ShapeDtypeStruct((NC, NPAD, D), _f32),
      mesh=_mesh(),
      scratch_types=[
          pltpu.VMEM((CW, WIN), jnp.int32),
          pltpu.VMEM((CW, WIN), jnp.int32),
          pltpu.VMEM((WIN, D), _f32),
          pltpu.VMEM((WIN, D), _f32),
          pltpu.VMEM_SHARED((NPAD, D), _f32),
          pltpu.SemaphoreType.DMA,
          pltpu.SemaphoreType.DMA,
      ],
  )
  def prop(hs_hbm, src_hbm, dst_hbm, out_hbm,
           isrc, idst, rows0, rows1, acc, gsem0, gsem1):
    c = lax.axis_index("c")
    s = lax.axis_index("s")
    wid = s * NC + c
    r0 = s * RPT

    def zbody(r, carry):
      for j in range(D // LANES):
        rows0[r, pl.ds(j * LANES, LANES)] = jnp.zeros((LANES,), _f32)
      return carry

    lax.fori_loop(0, WIN, zbody, 0)
    for k in range(RPT // WIN):
      pltpu.sync_copy(rows0, acc.at[pl.ds(r0 + k * WIN, WIN)])
    plsc.subcore_barrier()

    def chunk_body(ch, carry):
      pltpu.sync_copy(src_hbm.at[wid, pl.ds(ch * CW, CW)], isrc)
      pltpu.sync_copy(dst_hbm.at[wid, pl.ds(ch * CW, CW)], idst)
      pltpu.async_copy(hs_hbm.at[isrc.at[0]], rows0, gsem0)

      def body(w2, c2):
        w = 2 * w2
        pltpu.async_copy(hs_hbm.at[isrc.at[w + 1]], rows1, gsem1)
        pltpu.make_async_copy(hs_hbm.at[isrc.at[w]], rows0, gsem0).wait()
        pltpu.sync_copy(rows0, acc.at[idst.at[w]], add=True)

        @pl.when(w2 + 1 < CW // 2)
        def _():
          pltpu.async_copy(hs_hbm.at[isrc.at[w + 2]], rows0, gsem0)

        pltpu.make_async_copy(hs_hbm.at[isrc.at[w + 1]], rows1, gsem1).wait()
        pltpu.sync_copy(rows1, acc.at[idst.at[w + 1]], add=True)
        return c2

      lax.fori_loop(0, CW // 2, body, 0)
      return carry

    lax.fori_loop(0, NCH, chunk_body, 0)
    plsc.subcore_barrier()
    pltpu.sync_copy(acc.at[pl.ds(r0, RPT)], out_hbm.at[c, pl.ds(r0, RPT)])

  return prop


def _row_block(N):
  for g in (8, 5, 4, 2, 1):
    if N % g == 0 and (N // g) % 8 == 0:
      return N // g
  return N


def _mm_body(x, w, h_o):
  h_o[...] = jnp.dot(x[...], w[...], preferred_element_type=_f32)


def _tc1_body(d0, d1, h, dinv_o, hs_o):
  dv = lax.rsqrt(d0[...] + d1[...] + 1.0)
  dinv_o[...] = dv
  hs_o[...] = h[...] * dv


def _tc2_body(a0, a1, hs, dv, w, hs2_o):
  g = (a0[...] + a1[...] + hs[...]) * dv[...]
  g2 = g * g
  hs2_o[...] = jnp.dot(g2, w[...], preferred_element_type=_f32) * dv[...]


def _tc3_body(a0, a1, hs, dv, out_o):
  out_o[...] = (a0[...] + a1[...] + hs[...]) * dv[...]


def kernel(x, edge_index, W1, W2):
  N, D = x.shape
  E = edge_index.shape[1]

  NWIN = -(-E // (NW * WIN))
  NWIN += NWIN % 2
  EPAD = NW * NWIN * WIN
  RPT = (-(-(N + 1) // NS) + 127) // 128 * 128
  NPAD = NS * RPT
  GR = NPAD - N

  src = edge_index[0].astype(jnp.int32)
  dst = edge_index[1].astype(jnp.int32)
  pad = EPAD - E
  padi = jnp.arange(pad, dtype=jnp.int32)
  src3 = jnp.concatenate([src, padi % N]).reshape(NW, NWIN, WIN)
  dst3 = jnp.concatenate([dst, N + padi % GR]).reshape(NW, NWIN, WIN)

  degflat = _make_deg(NPAD, NWIN)(dst3)
  d0 = degflat[:N, None]
  d1 = degflat[NPAD:NPAD + N, None]

  BLK = _row_block(N)
  G = N // BLK
  colspec = pl.BlockSpec((BLK, 1), lambda i: (i, 0))
  matspec = pl.BlockSpec((BLK, D), lambda i: (i, 0))
  wspec = pl.BlockSpec((D, D), lambda i: (0, 0))

  h1 = pl.pallas_call(
      _mm_body,
      grid=(G,),
      in_specs=[matspec, wspec],
      out_specs=matspec,
      out_shape=jax.ShapeDtypeStruct((N, D), _f32),
  )(x, W1)

  dinv, hs1 = pl.pallas_call(
      _tc1_body,
      grid=(G,),
      in_specs=[colspec, colspec, matspec],
      out_specs=[colspec, matspec],
      out_shape=[
          jax.ShapeDtypeStruct((N, 1), _f32),
          jax.ShapeDtypeStruct((N, D), _f32),
      ],
  )(d0, d1, h1)

  prop = _make_prop(D, NPAD, NWIN)

  acc1 = prop(hs1, src3, dst3)
  hs2 = pl.pallas_call(
      _tc2_body,
      grid=(G,),
      in_specs=[matspec, matspec, matspec, colspec, wspec],
      out_specs=matspec,
      out_shape=jax.ShapeDtypeStruct((N, D), _f32),
  )(acc1[0, :N], acc1[1, :N], hs1, dinv, W2)

  acc2 = prop(hs2, src3, dst3)
  out = pl.pallas_call(
      _tc3_body,
      grid=(G,),
      in_specs=[matspec, matspec, matspec, colspec],
      out_specs=matspec,
      out_shape=jax.ShapeDtypeStruct((N, D), _f32),
  )(acc2[0, :N], acc2[1, :N], hs2, dinv)
  return out


# speedup vs baseline: 1.2586x; 1.0024x over previous
import functools

import jax
import jax.numpy as jnp
from jax import lax
from jax.experimental import pallas as pl
from jax.experimental.pallas import tpu as pltpu
from jax.experimental.pallas import tpu_sc as plsc

NC = 2
NS = 16
NW = NC * NS
LANES = 16
WIN = 128

_f32 = jnp.float32


def _mesh():
  return plsc.VectorSubcoreMesh(core_axis_name="c", subcore_axis_name="s")


def _make_deg(NPAD, NWIN):
  RPT = NPAD // NS

  @functools.partial(
      pl.kernel,
      out_type=jax.ShapeDtypeStruct((NC * NPAD,), _f32),
      mesh=_mesh(),
      scratch_types=[
          pltpu.VMEM((NWIN, WIN), jnp.int32),
          pltpu.VMEM((WIN,), _f32),
          pltpu.VMEM((RPT,), _f32),
          pltpu.VMEM_SHARED((NPAD,), _f32),
      ],
  )
  def deg(dst_hbm, out_hbm, idst, ones_v, zv, dacc):
    c = lax.axis_index("c")
    s = lax.axis_index("s")
    wid = s * NC + c
    for j in range(WIN // LANES):
      ones_v[pl.ds(j * LANES, LANES)] = jnp.ones((LANES,), _f32)
    for j in range(RPT // LANES):
      zv[pl.ds(j * LANES, LANES)] = jnp.zeros((LANES,), _f32)
    pltpu.sync_copy(zv, dacc.at[pl.ds(s * RPT, RPT)])
    pltpu.sync_copy(dst_hbm.at[wid], idst)
    plsc.subcore_barrier()

    def body(w, carry):
      pltpu.sync_copy(ones_v, dacc.at[idst.at[w]], add=True)
      return carry

    lax.fori_loop(0, NWIN, body, 0)
    plsc.subcore_barrier()
    pltpu.sync_copy(dacc.at[pl.ds(s * RPT, RPT)],
                    out_hbm.at[pl.ds(c * NPAD + s * RPT, RPT)])

  return deg


def _make_prop(D, NPAD, NWIN):
  RPT = NPAD // NS
  NCH = 2
  CW = NWIN // NCH

  @functools.partial(
      pl.kernel,
      out_type=jax.ShapeDtypeStruct((NC, NPAD, D), _f32),
      mesh=_mesh(),
      scratch_types=[
          pltpu.VMEM((CW, WIN), jnp.int32),
          pltpu.VMEM((CW, WIN), jnp.int32),
          pltpu.VMEM((WIN, D), _f32),
          pltpu.VMEM((WIN, D), _f32),
          pltpu.VMEM_SHARED((NPAD, D), _f32),
          pltpu.SemaphoreType.DMA,
          pltpu.SemaphoreType.DMA,
      ],
  )
  def prop(hs_hbm, src_hbm, dst_hbm, out_hbm,
           isrc, idst, rows0, rows1, acc, gsem0, gsem1):
    c = lax.axis_index("c")
    s = lax.axis_index("s")
    wid = s * NC + c
    r0 = s * RPT

    def zbody(r, carry):
      for j in range(D // LANES):
        rows0[r, pl.ds(j * LANES, LANES)] = jnp.zeros((LANES,), _f32)
      return carry

    lax.fori_loop(0, WIN, zbody, 0)
    for k in range(RPT // WIN):
      pltpu.sync_copy(rows0, acc.at[pl.ds(r0 + k * WIN, WIN)])
    plsc.subcore_barrier()

    def chunk_body(ch, carry):
      pltpu.sync_copy(src_hbm.at[wid, pl.ds(ch * CW, CW)], isrc)
      pltpu.sync_copy(dst_hbm.at[wid, pl.ds(ch * CW, CW)], idst)
      pltpu.async_copy(hs_hbm.at[isrc.at[0]], rows0, gsem0)

      def body(w2, c2):
        w = 2 * w2
        pltpu.async_copy(hs_hbm.at[isrc.at[w + 1]], rows1, gsem1)
        pltpu.make_async_copy(hs_hbm.at[isrc.at[w]], rows0, gsem0).wait()
        pltpu.sync_copy(rows0, acc.at[idst.at[w]], add=True)

        @pl.when(w2 + 1 < CW // 2)
        def _():
          pltpu.async_copy(hs_hbm.at[isrc.at[w + 2]], rows0, gsem0)

        pltpu.make_async_copy(hs_hbm.at[isrc.at[w + 1]], rows1, gsem1).wait()
        pltpu.sync_copy(rows1, acc.at[idst.at[w + 1]], add=True)
        return c2

      lax.fori_loop(0, CW // 2, body, 0)
      return carry

    lax.fori_loop(0, NCH, chunk_body, 0)
    plsc.subcore_barrier()
    pltpu.sync_copy(acc.at[pl.ds(r0, RPT)], out_hbm.at[c, pl.ds(r0, RPT)])

  return prop


def _row_block(N):
  for g in (8, 5, 4, 2, 1):
    if N % g == 0 and (N // g) % 8 == 0:
      return N // g
  return N


def _mm_body(x, w, h_o):
  h_o[...] = jnp.dot(x[...], w[...], preferred_element_type=_f32)


def _tc1_body(d0, d1, h, dinv_o, hs_o):
  dv = lax.rsqrt(d0[...] + d1[...] + 1.0)
  dinv_o[...] = dv
  hs_o[...] = h[...] * dv


def _tc2_body(a0, a1, hs, dv, w, hs2_o):
  g = (a0[...] + a1[...] + hs[...]) * dv[...]
  g2 = g * g
  hs2_o[...] = jnp.dot(g2, w[...], preferred_element_type=_f32) * dv[...]


def _tc3_body(a0, a1, hs, dv, out_o):
  out_o[...] = (a0[...] + a1[...] + hs[...]) * dv[...]


def kernel(x, edge_index, W1, W2):
  N, D = x.shape
  E = edge_index.shape[1]

  NWIN = -(-E // (NW * WIN))
  NWIN += (-NWIN) % 16
  EPAD = NW * NWIN * WIN
  RPT = (-(-(N + 1) // NS) + 127) // 128 * 128
  NPAD = NS * RPT
  GR = NPAD - N

  src = edge_index[0].astype(jnp.int32)
  dst = edge_index[1].astype(jnp.int32)
  pad = EPAD - E
  padi = jnp.arange(pad, dtype=jnp.int32)
  src3 = jnp.concatenate([src, padi % N]).reshape(NW, NWIN, WIN)
  dst3 = jnp.concatenate([dst, N + padi % GR]).reshape(NW, NWIN, WIN)

  degflat = _make_deg(NPAD, NWIN)(dst3)
  d0 = degflat[:N, None]
  d1 = degflat[NPAD:NPAD + N, None]

  BLK = _row_block(N)
  G = N // BLK
  colspec = pl.BlockSpec((BLK, 1), lambda i: (i, 0))
  matspec = pl.BlockSpec((BLK, D), lambda i: (i, 0))
  wspec = pl.BlockSpec((D, D), lambda i: (0, 0))

  h1 = pl.pallas_call(
      _mm_body,
      grid=(G,),
      in_specs=[matspec, wspec],
      out_specs=matspec,
      out_shape=jax.ShapeDtypeStruct((N, D), _f32),
  )(x, W1)

  dinv, hs1 = pl.pallas_call(
      _tc1_body,
      grid=(G,),
      in_specs=[colspec, colspec, matspec],
      out_specs=[colspec, matspec],
      out_shape=[
          jax.ShapeDtypeStruct((N, 1), _f32),
          jax.ShapeDtypeStruct((N, D), _f32),
      ],
  )(d0, d1, h1)

  prop = _make_prop(D, NPAD, NWIN)

  acc1 = prop(hs1, src3, dst3)
  hs2 = pl.pallas_call(
      _tc2_body,
      grid=(G,),
      in_specs=[matspec, matspec, matspec, colspec, wspec],
      out_specs=matspec,
      out_shape=jax.ShapeDtypeStruct((N, D), _f32),
  )(acc1[0, :N], acc1[1, :N], hs1, dinv, W2)

  acc2 = prop(hs2, src3, dst3)
  out = pl.pallas_call(
      _tc3_body,
      grid=(G,),
      in_specs=[matspec, matspec, matspec, colspec],
      out_specs=matspec,
      out_shape=jax.ShapeDtypeStruct((N, D), _f32),
  )(acc2[0, :N], acc2[1, :N], hs2, dinv)
  return out


# speedup vs baseline: 1.3137x; 1.0438x over previous
import functools

import jax
import jax.numpy as jnp
from jax import lax
from jax.experimental import pallas as pl
from jax.experimental.pallas import tpu as pltpu
from jax.experimental.pallas import tpu_sc as plsc

NC = 2
NS = 16
NW = NC * NS
LANES = 16
WIN = 128

_f32 = jnp.float32


def _mesh():
  return plsc.VectorSubcoreMesh(core_axis_name="c", subcore_axis_name="s")


def _make_deg(NPAD, NWIN):
  RPT = NPAD // NS

  @functools.partial(
      pl.kernel,
      out_type=jax.ShapeDtypeStruct((NC * NPAD,), _f32),
      mesh=_mesh(),
      scratch_types=[
          pltpu.VMEM((NWIN, WIN), jnp.int32),
          pltpu.VMEM((WIN,), _f32),
          pltpu.VMEM((RPT,), _f32),
          pltpu.VMEM_SHARED((NPAD,), _f32),
      ],
  )
  def deg(dst_hbm, out_hbm, idst, ones_v, zv, dacc):
    c = lax.axis_index("c")
    s = lax.axis_index("s")
    wid = s * NC + c
    for j in range(WIN // LANES):
      ones_v[pl.ds(j * LANES, LANES)] = jnp.ones((LANES,), _f32)
    for j in range(RPT // LANES):
      zv[pl.ds(j * LANES, LANES)] = jnp.zeros((LANES,), _f32)
    pltpu.sync_copy(zv, dacc.at[pl.ds(s * RPT, RPT)])
    pltpu.sync_copy(dst_hbm.at[wid], idst)
    plsc.subcore_barrier()

    def body(w, carry):
      pltpu.sync_copy(ones_v, dacc.at[idst.at[w]], add=True)
      return carry

    lax.fori_loop(0, NWIN, body, 0)
    plsc.subcore_barrier()
    pltpu.sync_copy(dacc.at[pl.ds(s * RPT, RPT)],
                    out_hbm.at[pl.ds(c * NPAD + s * RPT, RPT)])

  return deg


def _make_prop(D, NPAD, NWIN):
  RPT = NPAD // NS
  NCH = 2
  CW = NWIN // NCH

  @functools.partial(
      pl.kernel,
      out_type=jax.ShapeDtypeStruct((NC, NPAD, D), _f32),
      mesh=_mesh(),
      scratch_types=[
          pltpu.VMEM((CW, WIN), jnp.int32),
          pltpu.VMEM((CW, WIN), jnp.int32),
          pltpu.VMEM((WIN, D), _f32),
          pltpu.VMEM((WIN, D), _f32),
          pltpu.VMEM_SHARED((NPAD, D), _f32),
          pltpu.SemaphoreType.DMA,
          pltpu.SemaphoreType.DMA,
      ],
  )
  def prop(hs_hbm, src_hbm, dst_hbm, out_hbm,
           isrc, idst, rows0, rows1, acc, gsem0, gsem1):
    c = lax.axis_index("c")
    s = lax.axis_index("s")
    wid = s * NC + c
    r0 = s * RPT

    def zbody(r, carry):
      for j in range(D // LANES):
        rows0[r, pl.ds(j * LANES, LANES)] = jnp.zeros((LANES,), _f32)
      return carry

    lax.fori_loop(0, WIN, zbody, 0)
    for k in range(RPT // WIN):
      pltpu.sync_copy(rows0, acc.at[pl.ds(r0 + k * WIN, WIN)])
    plsc.subcore_barrier()

    def chunk_body(ch, carry):
      pltpu.sync_copy(src_hbm.at[wid, pl.ds(ch * CW, CW)], isrc)
      pltpu.sync_copy(dst_hbm.at[wid, pl.ds(ch * CW, CW)], idst)
      pltpu.async_copy(hs_hbm.at[isrc.at[0]], rows0, gsem0)

      def body(w2, c2):
        w = 2 * w2
        pltpu.async_copy(hs_hbm.at[isrc.at[w + 1]], rows1, gsem1)
        pltpu.make_async_copy(hs_hbm.at[isrc.at[w]], rows0, gsem0).wait()
        pltpu.sync_copy(rows0, acc.at[idst.at[w]], add=True)

        @pl.when(w2 + 1 < CW // 2)
        def _():
          pltpu.async_copy(hs_hbm.at[isrc.at[w + 2]], rows0, gsem0)

        pltpu.make_async_copy(hs_hbm.at[isrc.at[w + 1]], rows1, gsem1).wait()
        pltpu.sync_copy(rows1, acc.at[idst.at[w + 1]], add=True)
        return c2

      lax.fori_loop(0, CW // 2, body, 0)
      return carry

    lax.fori_loop(0, NCH, chunk_body, 0)
    plsc.subcore_barrier()
    pltpu.sync_copy(acc.at[pl.ds(r0, RPT)], out_hbm.at[c, pl.ds(r0, RPT)])

  return prop


def _row_block(N):
  for g in (8, 5, 4, 2, 1):
    if N % g == 0 and (N // g) % 8 == 0:
      return N // g
  return N


def _mm_body(x, w, h_o):
  h_o[...] = jnp.dot(x[...], w[...], preferred_element_type=_f32)


def _tc1_body(d0, d1, h, dinv_o, hs_o):
  dv = lax.rsqrt(d0[...] + d1[...] + 1.0)
  dinv_o[...] = dv
  hs_o[...] = h[...] * dv


def _tc2_body(a0, a1, hs, dv, w, hs2_o):
  g = (a0[0] + a1[0] + hs[...]) * dv[...]
  g2 = g * g
  hs2_o[...] = jnp.dot(g2, w[...], preferred_element_type=_f32) * dv[...]


def _tc3_body(a0, a1, hs, dv, out_o):
  out_o[...] = (a0[0] + a1[0] + hs[...]) * dv[...]


def kernel(x, edge_index, W1, W2):
  N, D = x.shape
  E = edge_index.shape[1]

  NWIN = -(-E // (NW * WIN))
  NWIN += (-NWIN) % 16
  EPAD = NW * NWIN * WIN
  RPT = (-(-(N + 1) // NS) + 127) // 128 * 128
  NPAD = NS * RPT
  GR = NPAD - N

  src = edge_index[0].astype(jnp.int32)
  dst = edge_index[1].astype(jnp.int32)
  pad = EPAD - E
  padi = jnp.arange(pad, dtype=jnp.int32)
  src3 = jnp.concatenate([src, padi % N]).reshape(NW, NWIN, WIN)
  dst3 = jnp.concatenate([dst, N + padi % GR]).reshape(NW, NWIN, WIN)

  degflat = _make_deg(NPAD, NWIN)(dst3)
  d0 = degflat[:N, None]
  d1 = degflat[NPAD:NPAD + N, None]

  BLK = _row_block(N)
  G = N // BLK
  colspec = pl.BlockSpec((BLK, 1), lambda i: (i, 0))
  matspec = pl.BlockSpec((BLK, D), lambda i: (i, 0))
  wspec = pl.BlockSpec((D, D), lambda i: (0, 0))
  acc0spec = pl.BlockSpec((1, BLK, D), lambda i: (0, i, 0))
  acc1spec = pl.BlockSpec((1, BLK, D), lambda i: (1, i, 0))

  h1 = pl.pallas_call(
      _mm_body,
      grid=(G,),
      in_specs=[matspec, wspec],
      out_specs=matspec,
      out_shape=jax.ShapeDtypeStruct((N, D), _f32),
  )(x, W1)

  dinv, hs1 = pl.pallas_call(
      _tc1_body,
      grid=(G,),
      in_specs=[colspec, colspec, matspec],
      out_specs=[colspec, matspec],
      out_shape=[
          jax.ShapeDtypeStruct((N, 1), _f32),
          jax.ShapeDtypeStruct((N, D), _f32),
      ],
  )(d0, d1, h1)

  prop = _make_prop(D, NPAD, NWIN)

  acc1 = prop(hs1, src3, dst3)
  hs2 = pl.pallas_call(
      _tc2_body,
      grid=(G,),
      in_specs=[acc0spec, acc1spec, matspec, colspec, wspec],
      out_specs=matspec,
      out_shape=jax.ShapeDtypeStruct((N, D), _f32),
  )(acc1, acc1, hs1, dinv, W2)

  acc2 = prop(hs2, src3, dst3)
  out = pl.pallas_call(
      _tc3_body,
      grid=(G,),
      in_specs=[acc0spec, acc1spec, matspec, colspec],
      out_specs=matspec,
      out_shape=jax.ShapeDtypeStruct((N, D), _f32),
  )(acc2, acc2, hs2, dinv)
  return out
